# baseline (speedup 1.0000x reference)
import functools

import jax
import jax.numpy as jnp
import numpy as np
from jax import lax
from jax.experimental import pallas as pl
from jax.experimental.pallas import tpu as pltpu
from jax.experimental.pallas import tpu_sc as plsc

VOCAB = 1000000
BIGRAM_DIM = 64
MODEL_DIM = 1024
BATCH = 4
SEQ = 4096
N_TOK = BATCH * SEQ

NC = 2
NS = 16
NW = NC * NS
CHUNK = N_TOK // NW
GROUPS = CHUNK // 16
IDX_ROWS = CHUNK // 128

PACK = 32

_MULT_CUR = np.int32(36313)
_MULT_PREV = np.int32(27191)
_MOD = np.int32(VOCAB - 1)

_SEL_EVEN = np.zeros((BIGRAM_DIM, PACK), np.float32)
_SEL_ODD = np.zeros((BIGRAM_DIM, PACK), np.float32)
for _j in range(PACK):
    _SEL_EVEN[2 * _j, _j] = 1.0
    _SEL_ODD[2 * _j + 1, _j] = 1.0


def _sc_hash_gather(tok_hbm, table_hbm, out_hbm, ext_v, idx2_v, par_v, rows_v,
                    half_v, sem):
    wid = lax.axis_index("s") * NC + lax.axis_index("c")
    base = wid * CHUNK
    is_rowstart = (base % SEQ) == 0

    ext_v[pl.ds(0, 16)] = jnp.zeros((16,), jnp.int32)

    @pl.when(is_rowstart)
    def _():
        pltpu.sync_copy(tok_hbm.at[pl.ds(base, CHUNK)], ext_v.at[pl.ds(8, CHUNK)])

    @pl.when(jnp.logical_not(is_rowstart))
    def _():
        pltpu.sync_copy(tok_hbm.at[pl.ds(base - 8, CHUNK + 8)], ext_v)

    lane = lax.iota(jnp.int32, 16)
    for i in range(GROUPS):
        cur = ext_v[pl.ds(8 + 16 * i, 16)]
        prev = ext_v[pl.ds(7 + 16 * i, 16)]
        mixed = jnp.bitwise_xor(_MULT_CUR * cur, _MULT_PREV * prev)
        rest = lax.rem(mixed, _MOD)
        rest = jnp.where(rest < 0, rest + _MOD, rest)
        pos_in_row = (base + 16 * i + lane) % SEQ
        rest = jnp.where(pos_in_row == 0, _MOD, rest)
        idx2_v[i // 8, pl.ds((i % 8) * 16, 16)] = ((rest >> 13) << 11) + (rest & 2047)
        par_v[pl.ds(16 * i, 16)] = ((rest >> 11) & 3) * PACK

    copies = [
        pltpu.async_copy(
            table_hbm.at[idx2_v.at[j]], rows_v.at[pl.ds(j * 128, 128)], sem
        )
        for j in range(IDX_ROWS)
    ]
    for c in copies:
        c.wait()

    def pick(tg, carry):
        offs = par_v[pl.ds(tg * 16, 16)]
        for b in range(16):
            t = tg * 16 + b
            off = offs[b]
            for g in range(2):
                half_v[t, pl.ds(g * 16, 16)] = rows_v[t, pl.ds(off + g * 16, 16)]
        return carry

    lax.fori_loop(0, GROUPS, pick, 0)
    pltpu.sync_copy(half_v, out_hbm.at[pl.ds(base, CHUNK)])


_gather_call = functools.partial(
    pl.kernel,
    mesh=plsc.VectorSubcoreMesh(core_axis_name="c", subcore_axis_name="s"),
    out_type=jax.ShapeDtypeStruct((N_TOK, PACK), jnp.int32),
    scratch_types=[
        pltpu.VMEM((CHUNK + 8,), jnp.int32),
        pltpu.VMEM((IDX_ROWS, 128), jnp.int32),
        pltpu.VMEM((CHUNK,), jnp.int32),
        pltpu.VMEM((CHUNK, 128), jnp.int32),
        pltpu.VMEM((CHUNK, PACK), jnp.int32),
        pltpu.SemaphoreType.DMA,
    ],
    compiler_params=pltpu.CompilerParams(use_tc_tiling_on_sc=False),
)(_sc_hash_gather)


def _bf16_bits(t):
    u = lax.bitcast_convert_type(t, jnp.int32)
    return (u + np.int32(0x7FFF) + ((u >> 16) & 1)) >> 16


def _tr_body(x_ref, se_ref, so_ref, o_ref):
    t_even = lax.dot_general(
        x_ref[...], se_ref[...], (((0,), (0,)), ((), ())),
        preferred_element_type=jnp.float32,
    )
    t_odd = lax.dot_general(
        x_ref[...], so_ref[...], (((0,), (0,)), ((), ())),
        preferred_element_type=jnp.float32,
    )
    packed = _bf16_bits(t_even) | (_bf16_bits(t_odd) << 16)
    q = _BK // 4
    o_ref[...] = jnp.concatenate(
        [packed[0:q, :], packed[q : 2 * q, :], packed[2 * q : 3 * q, :],
         packed[3 * q :, :]],
        axis=1,
    )


_BK = 8192
_TR_GRID = -(-VOCAB // _BK)
PACKED_ROWS = _TR_GRID * (_BK // 4)


def _untranspose_table(table_t):
    return pl.pallas_call(
        _tr_body,
        grid=(_TR_GRID,),
        in_specs=[
            pl.BlockSpec((BIGRAM_DIM, _BK), lambda i: (0, i)),
            pl.BlockSpec((BIGRAM_DIM, PACK), lambda i: (0, 0)),
            pl.BlockSpec((BIGRAM_DIM, PACK), lambda i: (0, 0)),
        ],
        out_specs=pl.BlockSpec((_BK // 4, 4 * PACK), lambda i: (i, 0)),
        out_shape=jax.ShapeDtypeStruct((PACKED_ROWS, 4 * PACK), jnp.int32),
    )(table_t, jnp.asarray(_SEL_EVEN), jnp.asarray(_SEL_ODD))


def _mm_body(x_ref, we_ref, wo_ref, s_ref, o_ref):
    u = x_ref[...]
    f_even = lax.bitcast_convert_type(u << 16, jnp.float32)
    f_odd = lax.bitcast_convert_type(u & np.int32(-65536), jnp.float32)
    acc = jnp.dot(f_even, we_ref[...], preferred_element_type=jnp.float32)
    acc = acc + jnp.dot(f_odd, wo_ref[...], preferred_element_type=jnp.float32)
    o_ref[...] = acc * s_ref[0, 0]


_BM = 1024


def _projection(gathered, w_even, w_odd, scale_arr):
    return pl.pallas_call(
        _mm_body,
        grid=(N_TOK // _BM,),
        in_specs=[
            pl.BlockSpec((_BM, PACK), lambda i: (i, 0)),
            pl.BlockSpec((PACK, MODEL_DIM), lambda i: (0, 0)),
            pl.BlockSpec((PACK, MODEL_DIM), lambda i: (0, 0)),
            pl.BlockSpec(memory_space=pltpu.SMEM),
        ],
        out_specs=pl.BlockSpec((_BM, MODEL_DIM), lambda i: (i, 0)),
        out_shape=jax.ShapeDtypeStruct((N_TOK, MODEL_DIM), jnp.float32),
    )(gathered, w_even, w_odd, scale_arr)


def kernel(token_ids, embed_table, proj_W, scale):
    tok = token_ids.astype(jnp.int32).reshape(N_TOK)
    table_packed = _untranspose_table(embed_table.T)
    gathered = _gather_call(tok, table_packed)
    scale_arr = jnp.reshape(scale.astype(jnp.float32), (1, 1))
    proj_wt = proj_W.T
    out = _projection(gathered, proj_wt[0::2, :], proj_wt[1::2, :], scale_arr)
    return out.reshape(BATCH, SEQ, MODEL_DIM)


# speedup vs baseline: 1.5936x; 1.5936x over previous
import functools

import jax
import jax.numpy as jnp
import numpy as np
from jax import lax
from jax.experimental import pallas as pl
from jax.experimental.pallas import tpu as pltpu
from jax.experimental.pallas import tpu_sc as plsc

VOCAB = 1000000
BIGRAM_DIM = 64
MODEL_DIM = 1024
BATCH = 4
SEQ = 4096
N_TOK = BATCH * SEQ

NC = 2
NS = 16
NW = NC * NS
CHUNK = N_TOK // NW
GROUPS = CHUNK // 16
IDX_ROWS = CHUNK // 128

PACK = 32

_MULT_CUR = np.int32(36313)
_MULT_PREV = np.int32(27191)
_MOD = np.int32(VOCAB - 1)

_SEL_EVEN = np.zeros((4 * BIGRAM_DIM, 4 * PACK), np.float32)
_SEL_ODD = np.zeros((4 * BIGRAM_DIM, 4 * PACK), np.float32)
for _q in range(4):
    for _j in range(PACK):
        _SEL_EVEN[_q * BIGRAM_DIM + 2 * _j, _q * PACK + _j] = 1.0
        _SEL_ODD[_q * BIGRAM_DIM + 2 * _j + 1, _q * PACK + _j] = 1.0


def _sc_hash_gather(tok_hbm, table_hbm, out_hbm, ext_v, idx2_v, par_v, rows_v,
                    half_v, sem):
    wid = lax.axis_index("s") * NC + lax.axis_index("c")
    base = wid * CHUNK
    is_rowstart = (base % SEQ) == 0

    ext_v[pl.ds(0, 16)] = jnp.zeros((16,), jnp.int32)

    @pl.when(is_rowstart)
    def _():
        pltpu.sync_copy(tok_hbm.at[pl.ds(base, CHUNK)], ext_v.at[pl.ds(8, CHUNK)])

    @pl.when(jnp.logical_not(is_rowstart))
    def _():
        pltpu.sync_copy(tok_hbm.at[pl.ds(base - 8, CHUNK + 8)], ext_v)

    lane = lax.iota(jnp.int32, 16)
    for i in range(GROUPS):
        cur = ext_v[pl.ds(8 + 16 * i, 16)]
        prev = ext_v[pl.ds(7 + 16 * i, 16)]
        mixed = jnp.bitwise_xor(_MULT_CUR * cur, _MULT_PREV * prev)
        rest = lax.rem(mixed, _MOD)
        rest = jnp.where(rest < 0, rest + _MOD, rest)
        pos_in_row = (base + 16 * i + lane) % SEQ
        rest = jnp.where(pos_in_row == 0, _MOD, rest)
        idx2_v[i // 8, pl.ds((i % 8) * 16, 16)] = ((rest >> 13) << 11) + (rest & 2047)
        par_v[pl.ds(16 * i, 16)] = ((rest >> 11) & 3) * PACK

    copies = [
        pltpu.async_copy(
            table_hbm.at[idx2_v.at[j]], rows_v.at[pl.ds(j * 128, 128)], sem
        )
        for j in range(IDX_ROWS)
    ]
    for c in copies:
        c.wait()

    def pick(tg, carry):
        offs = par_v[pl.ds(tg * 16, 16)]
        for b in range(16):
            t = tg * 16 + b
            off = offs[b]
            for g in range(2):
                half_v[t, pl.ds(g * 16, 16)] = rows_v[t, pl.ds(off + g * 16, 16)]
        return carry

    lax.fori_loop(0, GROUPS, pick, 0)
    pltpu.sync_copy(half_v, out_hbm.at[pl.ds(base, CHUNK)])


_gather_call = functools.partial(
    pl.kernel,
    mesh=plsc.VectorSubcoreMesh(core_axis_name="c", subcore_axis_name="s"),
    out_type=jax.ShapeDtypeStruct((N_TOK, PACK), jnp.int32),
    scratch_types=[
        pltpu.VMEM((CHUNK + 8,), jnp.int32),
        pltpu.VMEM((IDX_ROWS, 128), jnp.int32),
        pltpu.VMEM((CHUNK,), jnp.int32),
        pltpu.VMEM((CHUNK, 128), jnp.int32),
        pltpu.VMEM((CHUNK, PACK), jnp.int32),
        pltpu.SemaphoreType.DMA,
    ],
    compiler_params=pltpu.CompilerParams(use_tc_tiling_on_sc=False),
)(_sc_hash_gather)


def _bf16_bits(t):
    u = lax.bitcast_convert_type(t, jnp.int32)
    return (u + np.int32(0x7FFF) + ((u >> 16) & 1)) >> 16


def _tr_body(x_ref, se_ref, so_ref, o_ref):
    q = _BK // 4

    def compute(x):
        xs = jnp.concatenate(
            [x[:, 0:q], x[:, q : 2 * q], x[:, 2 * q : 3 * q], x[:, 3 * q :]],
            axis=0,
        )
        t_even = lax.dot_general(
            xs, se_ref[...], (((0,), (0,)), ((), ())),
            preferred_element_type=jnp.float32,
        )
        t_odd = lax.dot_general(
            xs, so_ref[...], (((0,), (0,)), ((), ())),
            preferred_element_type=jnp.float32,
        )
        return _bf16_bits(t_even) | (_bf16_bits(t_odd) << 16)

    i = pl.program_id(0)

    @pl.when(i != _TR_GRID - 1)
    def _():
        o_ref[...] = compute(x_ref[...])

    @pl.when(i == _TR_GRID - 1)
    def _():
        x = x_ref[...]
        u = lax.bitcast_convert_type(x, jnp.int32)
        bad = (u & np.int32(0x7F800000)) == np.int32(0x7F800000)
        o_ref[...] = compute(jnp.where(bad, jnp.float32(0.0), x))


_BK = 8192
_TR_GRID = -(-VOCAB // _BK)
PACKED_ROWS = _TR_GRID * (_BK // 4)


def _untranspose_table(table_t):
    return pl.pallas_call(
        _tr_body,
        grid=(_TR_GRID,),
        in_specs=[
            pl.BlockSpec((BIGRAM_DIM, _BK), lambda i: (0, i)),
            pl.BlockSpec((4 * BIGRAM_DIM, 4 * PACK), lambda i: (0, 0)),
            pl.BlockSpec((4 * BIGRAM_DIM, 4 * PACK), lambda i: (0, 0)),
        ],
        out_specs=pl.BlockSpec((_BK // 4, 4 * PACK), lambda i: (i, 0)),
        out_shape=jax.ShapeDtypeStruct((PACKED_ROWS, 4 * PACK), jnp.int32),
    )(table_t, jnp.asarray(_SEL_EVEN), jnp.asarray(_SEL_ODD))


def _mm_body(x_ref, we_ref, wo_ref, s_ref, o_ref):
    u = x_ref[...]
    f_even = lax.bitcast_convert_type(u << 16, jnp.float32)
    f_odd = lax.bitcast_convert_type(u & np.int32(-65536), jnp.float32)
    acc = jnp.dot(f_even, we_ref[...], preferred_element_type=jnp.float32)
    acc = acc + jnp.dot(f_odd, wo_ref[...], preferred_element_type=jnp.float32)
    o_ref[...] = acc * s_ref[0, 0]


_BM = 1024


def _projection(gathered, w_even, w_odd, scale_arr):
    return pl.pallas_call(
        _mm_body,
        grid=(N_TOK // _BM,),
        in_specs=[
            pl.BlockSpec((_BM, PACK), lambda i: (i, 0)),
            pl.BlockSpec((PACK, MODEL_DIM), lambda i: (0, 0)),
            pl.BlockSpec((PACK, MODEL_DIM), lambda i: (0, 0)),
            pl.BlockSpec(memory_space=pltpu.SMEM),
        ],
        out_specs=pl.BlockSpec((_BM, MODEL_DIM), lambda i: (i, 0)),
        out_shape=jax.ShapeDtypeStruct((N_TOK, MODEL_DIM), jnp.float32),
    )(gathered, w_even, w_odd, scale_arr)


def kernel(token_ids, embed_table, proj_W, scale):
    tok = token_ids.astype(jnp.int32).reshape(N_TOK)
    table_packed = _untranspose_table(embed_table.T)
    gathered = _gather_call(tok, table_packed)
    scale_arr = jnp.reshape(scale.astype(jnp.float32), (1, 1))
    proj_wt = proj_W.T
    out = _projection(gathered, proj_wt[0::2, :], proj_wt[1::2, :], scale_arr)
    return out.reshape(BATCH, SEQ, MODEL_DIM)


# speedup vs baseline: 1.6164x; 1.0143x over previous
import functools

import jax
import jax.numpy as jnp
import numpy as np
from jax import lax
from jax.experimental import pallas as pl
from jax.experimental.pallas import tpu as pltpu
from jax.experimental.pallas import tpu_sc as plsc

VOCAB = 1000000
BIGRAM_DIM = 64
MODEL_DIM = 1024
BATCH = 4
SEQ = 4096
N_TOK = BATCH * SEQ

NC = 2
NS = 16
NW = NC * NS
CHUNK = N_TOK // NW
GROUPS = CHUNK // 16
IDX_ROWS = CHUNK // 128

PACK = 32

_MULT_CUR = np.int32(36313)
_MULT_PREV = np.int32(27191)
_MOD = np.int32(VOCAB - 1)

_SEL_EVEN = np.zeros((4 * BIGRAM_DIM, 4 * PACK), np.float32)
_SEL_ODD = np.zeros((4 * BIGRAM_DIM, 4 * PACK), np.float32)
for _q in range(4):
    for _j in range(PACK):
        _SEL_EVEN[_q * BIGRAM_DIM + 2 * _j, _q * PACK + _j] = 1.0
        _SEL_ODD[_q * BIGRAM_DIM + 2 * _j + 1, _q * PACK + _j] = 1.0


def _sc_hash_gather(tok_hbm, table_hbm, out_hbm, ext_v, idx2_v, par_v, rows_v,
                    half_v, sem):
    wid = lax.axis_index("s") * NC + lax.axis_index("c")
    base = wid * CHUNK
    is_rowstart = (base % SEQ) == 0

    ext_v[pl.ds(0, 16)] = jnp.zeros((16,), jnp.int32)

    @pl.when(is_rowstart)
    def _():
        pltpu.sync_copy(tok_hbm.at[pl.ds(base, CHUNK)], ext_v.at[pl.ds(8, CHUNK)])

    @pl.when(jnp.logical_not(is_rowstart))
    def _():
        pltpu.sync_copy(tok_hbm.at[pl.ds(base - 8, CHUNK + 8)], ext_v)

    lane = lax.iota(jnp.int32, 16)
    for i in range(GROUPS):
        cur = ext_v[pl.ds(8 + 16 * i, 16)]
        prev = ext_v[pl.ds(7 + 16 * i, 16)]
        mixed = jnp.bitwise_xor(_MULT_CUR * cur, _MULT_PREV * prev)
        rest = lax.rem(mixed, _MOD)
        rest = jnp.where(rest < 0, rest + _MOD, rest)
        pos_in_row = (base + 16 * i + lane) % SEQ
        rest = jnp.where(pos_in_row == 0, _MOD, rest)
        idx2_v[i // 8, pl.ds((i % 8) * 16, 16)] = ((rest >> 13) << 11) + (rest & 2047)
        par_v[pl.ds(16 * i, 16)] = ((rest >> 11) & 3) * PACK

    copies = [
        pltpu.async_copy(
            table_hbm.at[idx2_v.at[j]], rows_v.at[pl.ds(j * 128, 128)], sem
        )
        for j in range(IDX_ROWS)
    ]
    for c in copies:
        c.wait()

    def pick(tg, carry):
        offs = par_v[pl.ds(tg * 16, 16)]
        for b in range(16):
            t = tg * 16 + b
            off = offs[b]
            for g in range(2):
                half_v[t, pl.ds(g * 16, 16)] = rows_v[t, pl.ds(off + g * 16, 16)]
        return carry

    lax.fori_loop(0, GROUPS, pick, 0)
    pltpu.sync_copy(half_v, out_hbm.at[pl.ds(base, CHUNK)])


_gather_call = functools.partial(
    pl.kernel,
    mesh=plsc.VectorSubcoreMesh(core_axis_name="c", subcore_axis_name="s"),
    out_type=jax.ShapeDtypeStruct((N_TOK, PACK), jnp.int32),
    scratch_types=[
        pltpu.VMEM((CHUNK + 8,), jnp.int32),
        pltpu.VMEM((IDX_ROWS, 128), jnp.int32),
        pltpu.VMEM((CHUNK,), jnp.int32),
        pltpu.VMEM((CHUNK, 128), jnp.int32),
        pltpu.VMEM((CHUNK, PACK), jnp.int32),
        pltpu.SemaphoreType.DMA,
    ],
    compiler_params=pltpu.CompilerParams(use_tc_tiling_on_sc=False),
)(_sc_hash_gather)


def _bf16_bits(t):
    u = lax.bitcast_convert_type(t, jnp.int32)
    return (u + np.int32(0x7FFF) + ((u >> 16) & 1)) >> 16


def _tr_body(x_ref, se_ref, so_ref, o_ref):
    q = _BK // 4

    def compute(x):
        xs = jnp.concatenate(
            [x[:, 0:q], x[:, q : 2 * q], x[:, 2 * q : 3 * q], x[:, 3 * q :]],
            axis=0,
        )
        xs16 = xs.astype(jnp.bfloat16)
        t_even = lax.dot_general(
            xs16, se_ref[...], (((0,), (0,)), ((), ())),
            preferred_element_type=jnp.float32,
        )
        t_odd = lax.dot_general(
            xs16, so_ref[...], (((0,), (0,)), ((), ())),
            preferred_element_type=jnp.float32,
        )
        u_e = lax.bitcast_convert_type(t_even, jnp.int32)
        u_o = lax.bitcast_convert_type(t_odd, jnp.int32)
        return lax.shift_right_logical(u_e, 16) | (u_o & np.int32(-65536))

    i = pl.program_id(0)

    @pl.when(i != _TR_GRID - 1)
    def _():
        o_ref[...] = compute(x_ref[...])

    @pl.when(i == _TR_GRID - 1)
    def _():
        x = x_ref[...]
        u = lax.bitcast_convert_type(x, jnp.int32)
        bad = (u & np.int32(0x7F800000)) == np.int32(0x7F800000)
        o_ref[...] = compute(jnp.where(bad, jnp.float32(0.0), x))


_BK = 8192
_TR_GRID = -(-VOCAB // _BK)
PACKED_ROWS = _TR_GRID * (_BK // 4)


def _untranspose_table(table_t):
    return pl.pallas_call(
        _tr_body,
        grid=(_TR_GRID,),
        in_specs=[
            pl.BlockSpec((BIGRAM_DIM, _BK), lambda i: (0, i)),
            pl.BlockSpec((4 * BIGRAM_DIM, 4 * PACK), lambda i: (0, 0)),
            pl.BlockSpec((4 * BIGRAM_DIM, 4 * PACK), lambda i: (0, 0)),
        ],
        out_specs=pl.BlockSpec((_BK // 4, 4 * PACK), lambda i: (i, 0)),
        out_shape=jax.ShapeDtypeStruct((PACKED_ROWS, 4 * PACK), jnp.int32),
    )(
        table_t,
        jnp.asarray(_SEL_EVEN, jnp.bfloat16),
        jnp.asarray(_SEL_ODD, jnp.bfloat16),
    )


def _mm_body(x_ref, we_ref, wo_ref, s_ref, o_ref):
    u = x_ref[...]
    f_even = lax.bitcast_convert_type(u << 16, jnp.float32)
    f_odd = lax.bitcast_convert_type(u & np.int32(-65536), jnp.float32)
    acc = jnp.dot(f_even, we_ref[...], preferred_element_type=jnp.float32)
    acc = acc + jnp.dot(f_odd, wo_ref[...], preferred_element_type=jnp.float32)
    o_ref[...] = acc * s_ref[0, 0]


_BM = 1024


def _projection(gathered, w_even, w_odd, scale_arr):
    return pl.pallas_call(
        _mm_body,
        grid=(N_TOK // _BM,),
        in_specs=[
            pl.BlockSpec((_BM, PACK), lambda i: (i, 0)),
            pl.BlockSpec((PACK, MODEL_DIM), lambda i: (0, 0)),
            pl.BlockSpec((PACK, MODEL_DIM), lambda i: (0, 0)),
            pl.BlockSpec(memory_space=pltpu.SMEM),
        ],
        out_specs=pl.BlockSpec((_BM, MODEL_DIM), lambda i: (i, 0)),
        out_shape=jax.ShapeDtypeStruct((N_TOK, MODEL_DIM), jnp.float32),
    )(gathered, w_even, w_odd, scale_arr)


def kernel(token_ids, embed_table, proj_W, scale):
    tok = token_ids.astype(jnp.int32).reshape(N_TOK)
    table_packed = _untranspose_table(embed_table.T)
    gathered = _gather_call(tok, table_packed)
    scale_arr = jnp.reshape(scale.astype(jnp.float32), (1, 1))
    proj_wt = proj_W.T
    out = _projection(gathered, proj_wt[0::2, :], proj_wt[1::2, :], scale_arr)
    return out.reshape(BATCH, SEQ, MODEL_DIM)


# speedup vs baseline: 1.9202x; 1.1880x over previous
import functools

import jax
import jax.numpy as jnp
import numpy as np
from jax import lax
from jax.experimental import pallas as pl
from jax.experimental.pallas import tpu as pltpu
from jax.experimental.pallas import tpu_sc as plsc

VOCAB = 1000000
BIGRAM_DIM = 64
MODEL_DIM = 1024
BATCH = 4
SEQ = 4096
N_TOK = BATCH * SEQ

NC = 2
NS = 16
NW = NC * NS
CHUNK = N_TOK // NW
GROUPS = CHUNK // 16
IDX_ROWS = CHUNK // 128

PACK = 32

_MULT_CUR = np.int32(36313)
_MULT_PREV = np.int32(27191)
_MOD = np.int32(VOCAB - 1)

_SEL_EVEN = np.zeros((4 * BIGRAM_DIM, 4 * PACK), np.float32)
_SEL_ODD = np.zeros((4 * BIGRAM_DIM, 4 * PACK), np.float32)
for _q in range(4):
    for _j in range(PACK):
        _SEL_EVEN[_q * BIGRAM_DIM + 2 * _j, _q * PACK + _j] = 1.0
        _SEL_ODD[_q * BIGRAM_DIM + 2 * _j + 1, _q * PACK + _j] = 1.0


def _sc_hash_gather(tok_hbm, table_hbm, out_hbm, ext_v, idx2_v, par_v, rows_v,
                    half_v, sem):
    wid = lax.axis_index("s") * NC + lax.axis_index("c")
    base = wid * CHUNK
    is_rowstart = (base % SEQ) == 0

    ext_v[pl.ds(0, 16)] = jnp.zeros((16,), jnp.int32)

    @pl.when(is_rowstart)
    def _():
        pltpu.sync_copy(tok_hbm.at[pl.ds(base, CHUNK)], ext_v.at[pl.ds(8, CHUNK)])

    @pl.when(jnp.logical_not(is_rowstart))
    def _():
        pltpu.sync_copy(tok_hbm.at[pl.ds(base - 8, CHUNK + 8)], ext_v)

    lane = lax.iota(jnp.int32, 16)
    for i in range(GROUPS):
        cur = ext_v[pl.ds(8 + 16 * i, 16)]
        prev = ext_v[pl.ds(7 + 16 * i, 16)]
        mixed = jnp.bitwise_xor(_MULT_CUR * cur, _MULT_PREV * prev)
        rest = lax.rem(mixed, _MOD)
        rest = jnp.where(rest < 0, rest + _MOD, rest)
        pos_in_row = (base + 16 * i + lane) % SEQ
        rest = jnp.where(pos_in_row == 0, _MOD, rest)
        idx2_v[i // 8, pl.ds((i % 8) * 16, 16)] = ((rest >> 14) << 12) + (rest & 4095)
        par_v[pl.ds(16 * i, 16)] = ((rest >> 12) & 3) * PACK

    copies = [
        pltpu.async_copy(
            table_hbm.at[idx2_v.at[j]], rows_v.at[pl.ds(j * 128, 128)], sem
        )
        for j in range(IDX_ROWS)
    ]
    for c in copies:
        c.wait()

    def pick(tg, carry):
        offs = par_v[pl.ds(tg * 16, 16)]
        for b in range(16):
            t = tg * 16 + b
            off = offs[b]
            for g in range(2):
                half_v[t, pl.ds(g * 16, 16)] = rows_v[t, pl.ds(off + g * 16, 16)]
        return carry

    lax.fori_loop(0, GROUPS, pick, 0)
    pltpu.sync_copy(half_v, out_hbm.at[pl.ds(base, CHUNK)])


_gather_call = functools.partial(
    pl.kernel,
    mesh=plsc.VectorSubcoreMesh(core_axis_name="c", subcore_axis_name="s"),
    out_type=jax.ShapeDtypeStruct((N_TOK, PACK), jnp.int32),
    scratch_types=[
        pltpu.VMEM((CHUNK + 8,), jnp.int32),
        pltpu.VMEM((IDX_ROWS, 128), jnp.int32),
        pltpu.VMEM((CHUNK,), jnp.int32),
        pltpu.VMEM((CHUNK, 128), jnp.int32),
        pltpu.VMEM((CHUNK, PACK), jnp.int32),
        pltpu.SemaphoreType.DMA,
    ],
    compiler_params=pltpu.CompilerParams(use_tc_tiling_on_sc=False),
)(_sc_hash_gather)


def _bf16_bits(t):
    u = lax.bitcast_convert_type(t, jnp.int32)
    return (u + np.int32(0x7FFF) + ((u >> 16) & 1)) >> 16


def _tr_body(x_ref, se_ref, so_ref, o_ref):
    q = _BK // 4

    def compute(x):
        xs = jnp.concatenate(
            [x[:, 0:q], x[:, q : 2 * q], x[:, 2 * q : 3 * q], x[:, 3 * q :]],
            axis=0,
        )
        xs16 = xs.astype(jnp.bfloat16)
        t_even = lax.dot_general(
            xs16, se_ref[...], (((0,), (0,)), ((), ())),
            preferred_element_type=jnp.float32,
        )
        t_odd = lax.dot_general(
            xs16, so_ref[...], (((0,), (0,)), ((), ())),
            preferred_element_type=jnp.float32,
        )
        u_e = lax.bitcast_convert_type(t_even, jnp.int32)
        u_o = lax.bitcast_convert_type(t_odd, jnp.int32)
        return lax.shift_right_logical(u_e, 16) | (u_o & np.int32(-65536))

    i = pl.program_id(0)

    @pl.when(i != _TR_GRID - 1)
    def _():
        o_ref[...] = compute(x_ref[...])

    @pl.when(i == _TR_GRID - 1)
    def _():
        x = x_ref[...]
        u = lax.bitcast_convert_type(x, jnp.int32)
        bad = (u & np.int32(0x7F800000)) == np.int32(0x7F800000)
        o_ref[...] = compute(jnp.where(bad, jnp.float32(0.0), x))


_BK = 16384
_TR_GRID = -(-VOCAB // _BK)
PACKED_ROWS = _TR_GRID * (_BK // 4)


def _untranspose_table(table_t):
    return pl.pallas_call(
        _tr_body,
        grid=(_TR_GRID,),
        in_specs=[
            pl.BlockSpec((BIGRAM_DIM, _BK), lambda i: (0, i)),
            pl.BlockSpec((4 * BIGRAM_DIM, 4 * PACK), lambda i: (0, 0)),
            pl.BlockSpec((4 * BIGRAM_DIM, 4 * PACK), lambda i: (0, 0)),
        ],
        out_specs=pl.BlockSpec((_BK // 4, 4 * PACK), lambda i: (i, 0)),
        out_shape=jax.ShapeDtypeStruct((PACKED_ROWS, 4 * PACK), jnp.int32),
    )(
        table_t,
        jnp.asarray(_SEL_EVEN, jnp.bfloat16),
        jnp.asarray(_SEL_ODD, jnp.bfloat16),
    )


def _mm_body(x_ref, we_ref, wo_ref, s_ref, o_ref):
    u = x_ref[...]
    f_even = lax.bitcast_convert_type(u << 16, jnp.float32)
    f_odd = lax.bitcast_convert_type(u & np.int32(-65536), jnp.float32)
    acc = jnp.dot(f_even, we_ref[...], preferred_element_type=jnp.float32)
    acc = acc + jnp.dot(f_odd, wo_ref[...], preferred_element_type=jnp.float32)
    o_ref[...] = acc * s_ref[0, 0]


_BM = 1024


def _projection(gathered, w_even, w_odd, scale_arr):
    return pl.pallas_call(
        _mm_body,
        grid=(N_TOK // _BM,),
        in_specs=[
            pl.BlockSpec((_BM, PACK), lambda i: (i, 0)),
            pl.BlockSpec((PACK, MODEL_DIM), lambda i: (0, 0)),
            pl.BlockSpec((PACK, MODEL_DIM), lambda i: (0, 0)),
            pl.BlockSpec(memory_space=pltpu.SMEM),
        ],
        out_specs=pl.BlockSpec((_BM, MODEL_DIM), lambda i: (i, 0)),
        out_shape=jax.ShapeDtypeStruct((N_TOK, MODEL_DIM), jnp.float32),
    )(gathered, w_even, w_odd, scale_arr)


def kernel(token_ids, embed_table, proj_W, scale):
    tok = token_ids.astype(jnp.int32).reshape(N_TOK)
    table_packed = _untranspose_table(embed_table.T)
    gathered = _gather_call(tok, table_packed)
    scale_arr = jnp.reshape(scale.astype(jnp.float32), (1, 1))
    proj_wt = proj_W.T
    out = _projection(gathered, proj_wt[0::2, :], proj_wt[1::2, :], scale_arr)
    return out.reshape(BATCH, SEQ, MODEL_DIM)


# speedup vs baseline: 2.0367x; 1.0607x over previous
import functools

import jax
import jax.numpy as jnp
import numpy as np
from jax import lax
from jax.experimental import pallas as pl
from jax.experimental.pallas import tpu as pltpu
from jax.experimental.pallas import tpu_sc as plsc

VOCAB = 1000000
BIGRAM_DIM = 64
MODEL_DIM = 1024
BATCH = 4
SEQ = 4096
N_TOK = BATCH * SEQ

NC = 2
NS = 16
NW = NC * NS
CHUNK = N_TOK // NW
GROUPS = CHUNK // 16
IDX_ROWS = CHUNK // 128

PACK = 32

_MULT_CUR = np.int32(36313)
_MULT_PREV = np.int32(27191)
_MOD = np.int32(VOCAB - 1)

_SEL_EVEN = np.zeros((4 * BIGRAM_DIM, 4 * PACK), np.float32)
_SEL_ODD = np.zeros((4 * BIGRAM_DIM, 4 * PACK), np.float32)
for _q in range(4):
    for _j in range(PACK):
        _SEL_EVEN[_q * BIGRAM_DIM + 2 * _j, _q * PACK + _j] = 1.0
        _SEL_ODD[_q * BIGRAM_DIM + 2 * _j + 1, _q * PACK + _j] = 1.0


def _sc_hash_gather(tok_hbm, table_hbm, out_hbm, ext_v, idx2_v, par_v, rows_v,
                    half_v, sem):
    wid = lax.axis_index("s") * NC + lax.axis_index("c")
    base = wid * CHUNK
    is_rowstart = (base % SEQ) == 0

    ext_v[pl.ds(0, 16)] = jnp.zeros((16,), jnp.int32)

    @pl.when(is_rowstart)
    def _():
        pltpu.sync_copy(tok_hbm.at[pl.ds(base, CHUNK)], ext_v.at[pl.ds(8, CHUNK)])

    @pl.when(jnp.logical_not(is_rowstart))
    def _():
        pltpu.sync_copy(tok_hbm.at[pl.ds(base - 8, CHUNK + 8)], ext_v)

    lane = lax.iota(jnp.int32, 16)
    for i in range(GROUPS):
        cur = ext_v[pl.ds(8 + 16 * i, 16)]
        prev = ext_v[pl.ds(7 + 16 * i, 16)]
        mixed = jnp.bitwise_xor(_MULT_CUR * cur, _MULT_PREV * prev)
        rest = lax.rem(mixed, _MOD)
        rest = jnp.where(rest < 0, rest + _MOD, rest)
        pos_in_row = (base + 16 * i + lane) % SEQ
        rest = jnp.where(pos_in_row == 0, _MOD, rest)
        idx2_v[i // 8, pl.ds((i % 8) * 16, 16)] = ((rest >> 15) << 13) + (rest & 8191)
        par_v[pl.ds(16 * i, 16)] = ((rest >> 13) & 3) * PACK

    copies = [
        pltpu.async_copy(
            table_hbm.at[idx2_v.at[j]], rows_v.at[pl.ds(j * 128, 128)], sem
        )
        for j in range(IDX_ROWS)
    ]
    for c in copies:
        c.wait()

    def pick(tg, carry):
        offs = par_v[pl.ds(tg * 16, 16)]
        for b in range(16):
            t = tg * 16 + b
            off = offs[b]
            for g in range(2):
                half_v[t, pl.ds(g * 16, 16)] = rows_v[t, pl.ds(off + g * 16, 16)]
        return carry

    lax.fori_loop(0, GROUPS, pick, 0)
    pltpu.sync_copy(half_v, out_hbm.at[pl.ds(base, CHUNK)])


_gather_call = functools.partial(
    pl.kernel,
    mesh=plsc.VectorSubcoreMesh(core_axis_name="c", subcore_axis_name="s"),
    out_type=jax.ShapeDtypeStruct((N_TOK, PACK), jnp.int32),
    scratch_types=[
        pltpu.VMEM((CHUNK + 8,), jnp.int32),
        pltpu.VMEM((IDX_ROWS, 128), jnp.int32),
        pltpu.VMEM((CHUNK,), jnp.int32),
        pltpu.VMEM((CHUNK, 128), jnp.int32),
        pltpu.VMEM((CHUNK, PACK), jnp.int32),
        pltpu.SemaphoreType.DMA,
    ],
    compiler_params=pltpu.CompilerParams(use_tc_tiling_on_sc=False),
)(_sc_hash_gather)


def _bf16_bits(t):
    u = lax.bitcast_convert_type(t, jnp.int32)
    return (u + np.int32(0x7FFF) + ((u >> 16) & 1)) >> 16


def _tr_body(x_ref, se_ref, so_ref, o_ref):
    q = _BK // 4

    def compute(x):
        xs = jnp.concatenate(
            [x[:, 0:q], x[:, q : 2 * q], x[:, 2 * q : 3 * q], x[:, 3 * q :]],
            axis=0,
        )
        xs16 = xs.astype(jnp.bfloat16)
        t_even = lax.dot_general(
            xs16, se_ref[...], (((0,), (0,)), ((), ())),
            preferred_element_type=jnp.float32,
        )
        t_odd = lax.dot_general(
            xs16, so_ref[...], (((0,), (0,)), ((), ())),
            preferred_element_type=jnp.float32,
        )
        u_e = lax.bitcast_convert_type(t_even, jnp.int32)
        u_o = lax.bitcast_convert_type(t_odd, jnp.int32)
        return lax.shift_right_logical(u_e, 16) | (u_o & np.int32(-65536))

    i = pl.program_id(0)

    @pl.when(i != _TR_GRID - 1)
    def _():
        o_ref[...] = compute(x_ref[...])

    @pl.when(i == _TR_GRID - 1)
    def _():
        x = x_ref[...]
        u = lax.bitcast_convert_type(x, jnp.int32)
        bad = (u & np.int32(0x7F800000)) == np.int32(0x7F800000)
        o_ref[...] = compute(jnp.where(bad, jnp.float32(0.0), x))


_BK = 32768
_TR_GRID = -(-VOCAB // _BK)
PACKED_ROWS = _TR_GRID * (_BK // 4)


def _untranspose_table(table_t):
    return pl.pallas_call(
        _tr_body,
        grid=(_TR_GRID,),
        in_specs=[
            pl.BlockSpec((BIGRAM_DIM, _BK), lambda i: (0, i)),
            pl.BlockSpec((4 * BIGRAM_DIM, 4 * PACK), lambda i: (0, 0)),
            pl.BlockSpec((4 * BIGRAM_DIM, 4 * PACK), lambda i: (0, 0)),
        ],
        out_specs=pl.BlockSpec((_BK // 4, 4 * PACK), lambda i: (i, 0)),
        out_shape=jax.ShapeDtypeStruct((PACKED_ROWS, 4 * PACK), jnp.int32),
    )(
        table_t,
        jnp.asarray(_SEL_EVEN, jnp.bfloat16),
        jnp.asarray(_SEL_ODD, jnp.bfloat16),
    )


def _mm_body(x_ref, we_ref, wo_ref, s_ref, o_ref):
    u = x_ref[...]
    f_even = lax.bitcast_convert_type(u << 16, jnp.float32)
    f_odd = lax.bitcast_convert_type(u & np.int32(-65536), jnp.float32)
    acc = jnp.dot(f_even, we_ref[...], preferred_element_type=jnp.float32)
    acc = acc + jnp.dot(f_odd, wo_ref[...], preferred_element_type=jnp.float32)
    o_ref[...] = acc * s_ref[0, 0]


_BM = 1024


def _projection(gathered, w_even, w_odd, scale_arr):
    return pl.pallas_call(
        _mm_body,
        grid=(N_TOK // _BM,),
        in_specs=[
            pl.BlockSpec((_BM, PACK), lambda i: (i, 0)),
            pl.BlockSpec((PACK, MODEL_DIM), lambda i: (0, 0)),
            pl.BlockSpec((PACK, MODEL_DIM), lambda i: (0, 0)),
            pl.BlockSpec(memory_space=pltpu.SMEM),
        ],
        out_specs=pl.BlockSpec((_BM, MODEL_DIM), lambda i: (i, 0)),
        out_shape=jax.ShapeDtypeStruct((N_TOK, MODEL_DIM), jnp.float32),
    )(gathered, w_even, w_odd, scale_arr)


def kernel(token_ids, embed_table, proj_W, scale):
    tok = token_ids.astype(jnp.int32).reshape(N_TOK)
    table_packed = _untranspose_table(embed_table.T)
    gathered = _gather_call(tok, table_packed)
    scale_arr = jnp.reshape(scale.astype(jnp.float32), (1, 1))
    proj_wt = proj_W.T
    out = _projection(gathered, proj_wt[0::2, :], proj_wt[1::2, :], scale_arr)
    return out.reshape(BATCH, SEQ, MODEL_DIM)
